# Initial kernel scaffold; baseline (speedup 1.0000x reference)
import functools

import jax
import jax.numpy as jnp
from jax import lax
from jax.experimental import pallas as pl
from jax.experimental.pallas import tpu as pltpu
from jax.experimental.pallas import tpu_sc as plsc

N_NODES = 10000
N_EDGES = 320000
EMB = 128

NC = 2
NS = 16
NW = NC * NS

EPT = N_EDGES // NW
K = 80
NCHUNK = EPT // K
RPT = N_NODES // NS
ZR = 125


def _mlp_body(x_ref, w1_ref, b1_ref, w2_ref, b2_ref, o_ref):
    h = jnp.dot(x_ref[...], w1_ref[...], preferred_element_type=jnp.float32)
    h = jnp.maximum(h + b1_ref[...], 0.0)
    h = jnp.dot(h, w2_ref[...], preferred_element_type=jnp.float32)
    o_ref[...] = jnp.maximum(h + b2_ref[...], 0.0)


def _mlp(X, W1, b1, W2, b2):
    blk = 1000
    grid = (N_NODES // blk,)
    return pl.pallas_call(
        _mlp_body,
        grid=grid,
        in_specs=[
            pl.BlockSpec((blk, EMB), lambda i: (i, 0)),
            pl.BlockSpec((EMB, EMB), lambda i: (0, 0)),
            pl.BlockSpec((1, EMB), lambda i: (0, 0)),
            pl.BlockSpec((EMB, EMB), lambda i: (0, 0)),
            pl.BlockSpec((1, EMB), lambda i: (0, 0)),
        ],
        out_specs=pl.BlockSpec((blk, EMB), lambda i: (i, 0)),
        out_shape=jax.ShapeDtypeStruct((N_NODES, EMB), jnp.float32),
    )(X, W1, b1, W2, b2)


@functools.partial(
    pl.kernel,
    mesh=plsc.VectorSubcoreMesh(core_axis_name="c", subcore_axis_name="s"),
    out_type=jax.ShapeDtypeStruct((NC, N_NODES, EMB), jnp.float32),
    scratch_types=[
        pltpu.VMEM((K,), jnp.int32),
        pltpu.VMEM((K,), jnp.int32),
        pltpu.VMEM((K, EMB), jnp.float32),
        pltpu.VMEM((ZR, EMB), jnp.float32),
        pltpu.VMEM_SHARED((N_NODES, EMB), jnp.float32),
        pltpu.SemaphoreType.DMA,
    ],
)
def _scatter_kernel(h_hbm, src_hbm, dst_hbm, out_hbm,
                    src_v, dst_v, rows_v, stage_v, acc, sem):
    c = lax.axis_index("c")
    s = lax.axis_index("s")
    wid = c * NS + s

    zero16 = jnp.zeros((16,), jnp.float32)

    def zero_row(r, carry):
        for j in range(EMB // 16):
            stage_v[r, pl.ds(j * 16, 16)] = zero16
        return carry

    lax.fori_loop(0, ZR, zero_row, 0)
    for j in range(RPT // ZR):
        pltpu.sync_copy(stage_v, acc.at[pl.ds(s * RPT + j * ZR, ZR)])
    plsc.subcore_barrier()

    ebase = wid * EPT

    def chunk(i, carry):
        b = ebase + i * K
        pltpu.sync_copy(src_hbm.at[pl.ds(b, K)], src_v)
        pltpu.sync_copy(dst_hbm.at[pl.ds(b, K)], dst_v)
        pltpu.async_copy(h_hbm.at[src_v], rows_v, sem).wait()
        pltpu.sync_copy(rows_v, acc.at[dst_v], add=True)
        return carry

    lax.fori_loop(0, NCHUNK, chunk, 0)
    plsc.subcore_barrier()

    for j in range(RPT // ZR):
        r0 = s * RPT + j * ZR
        pltpu.sync_copy(acc.at[pl.ds(r0, ZR)], stage_v)
        pltpu.sync_copy(stage_v, out_hbm.at[c, pl.ds(r0, ZR)])


def _add_body(p_ref, o_ref):
    o_ref[...] = p_ref[0] + p_ref[1]


def _combine(partials):
    blk = 1000
    grid = (N_NODES // blk,)
    return pl.pallas_call(
        _add_body,
        grid=grid,
        in_specs=[pl.BlockSpec((NC, blk, EMB), lambda i: (0, i, 0))],
        out_specs=pl.BlockSpec((blk, EMB), lambda i: (i, 0)),
        out_shape=jax.ShapeDtypeStruct((N_NODES, EMB), jnp.float32),
    )(partials)


def kernel(X, edge_index, W1, b1, W2, b2):
    h = _mlp(X, W1, b1.reshape(1, EMB), W2, b2.reshape(1, EMB))
    src = edge_index[0].astype(jnp.int32)
    dst = edge_index[1].astype(jnp.int32)
    partials = _scatter_kernel(h, src, dst)
    return _combine(partials)


# speedup vs baseline: 5.4100x; 5.4100x over previous
import functools

import jax
import jax.numpy as jnp
from jax import lax
from jax.experimental import pallas as pl
from jax.experimental.pallas import tpu as pltpu
from jax.experimental.pallas import tpu_sc as plsc

N_NODES = 10000
N_EDGES = 320000
EMB = 128

NC = 2
NS = 16
NW = NC * NS

EPT = N_EDGES // NW
K = 80
NCHUNK = EPT // K
N_PAD = 10240
RPT = N_PAD // NS
ZR = 128


def _mlp_body(x_ref, w1_ref, b1_ref, w2_ref, b2_ref, o_ref):
    h = jnp.dot(x_ref[...], w1_ref[...], preferred_element_type=jnp.float32)
    h = jnp.maximum(h + b1_ref[...], 0.0)
    h = jnp.dot(h, w2_ref[...], preferred_element_type=jnp.float32)
    o_ref[...] = jnp.maximum(h + b2_ref[...], 0.0)


def _mlp(X, W1, b1, W2, b2):
    blk = 1000
    grid = (N_NODES // blk,)
    return pl.pallas_call(
        _mlp_body,
        grid=grid,
        in_specs=[
            pl.BlockSpec((blk, EMB), lambda i: (i, 0)),
            pl.BlockSpec((EMB, EMB), lambda i: (0, 0)),
            pl.BlockSpec((1, EMB), lambda i: (0, 0)),
            pl.BlockSpec((EMB, EMB), lambda i: (0, 0)),
            pl.BlockSpec((1, EMB), lambda i: (0, 0)),
        ],
        out_specs=pl.BlockSpec((blk, EMB), lambda i: (i, 0)),
        out_shape=jax.ShapeDtypeStruct((N_NODES, EMB), jnp.float32),
    )(X, W1, b1, W2, b2)


@functools.partial(
    pl.kernel,
    mesh=plsc.VectorSubcoreMesh(core_axis_name="c", subcore_axis_name="s"),
    out_type=jax.ShapeDtypeStruct((NC, N_PAD, EMB), jnp.float32),
    scratch_types=[
        pltpu.VMEM((K,), jnp.int32),
        pltpu.VMEM((K,), jnp.int32),
        pltpu.VMEM((K, EMB), jnp.float32),
        pltpu.VMEM((ZR, EMB), jnp.float32),
        pltpu.VMEM_SHARED((N_PAD, EMB), jnp.float32),
        pltpu.SemaphoreType.DMA,
    ],
)
def _scatter_kernel(h_hbm, src_hbm, dst_hbm, out_hbm,
                    src_v, dst_v, rows_v, stage_v, acc, sem):
    c = lax.axis_index("c")
    s = lax.axis_index("s")
    wid = c * NS + s

    zero16 = jnp.zeros((16,), jnp.float32)

    def zero_row(r, carry):
        for j in range(EMB // 16):
            stage_v[r, pl.ds(j * 16, 16)] = zero16
        return carry

    lax.fori_loop(0, ZR, zero_row, 0)
    for j in range(RPT // ZR):
        pltpu.sync_copy(stage_v, acc.at[pl.ds(s * RPT + j * ZR, ZR)])
    plsc.subcore_barrier()

    ebase = wid * EPT

    def chunk(i, carry):
        b = ebase + i * K
        pltpu.sync_copy(src_hbm.at[pl.ds(b, K)], src_v)
        pltpu.sync_copy(dst_hbm.at[pl.ds(b, K)], dst_v)
        pltpu.async_copy(h_hbm.at[src_v], rows_v, sem).wait()
        pltpu.sync_copy(rows_v, acc.at[dst_v], add=True)
        return carry

    lax.fori_loop(0, NCHUNK, chunk, 0)
    plsc.subcore_barrier()

    for j in range(RPT // ZR):
        r0 = s * RPT + j * ZR
        pltpu.sync_copy(acc.at[pl.ds(r0, ZR)], stage_v)
        pltpu.sync_copy(stage_v, out_hbm.at[c, pl.ds(r0, ZR)])


def _add_body(p_ref, o_ref):
    o_ref[...] = p_ref[0] + p_ref[1]


def _combine(partials):
    blk = 1000
    grid = (N_NODES // blk,)
    return pl.pallas_call(
        _add_body,
        grid=grid,
        in_specs=[pl.BlockSpec((NC, blk, EMB), lambda i: (0, i, 0))],
        out_specs=pl.BlockSpec((blk, EMB), lambda i: (i, 0)),
        out_shape=jax.ShapeDtypeStruct((N_NODES, EMB), jnp.float32),
    )(partials)


def kernel(X, edge_index, W1, b1, W2, b2):
    h = _mlp(X, W1, b1.reshape(1, EMB), W2, b2.reshape(1, EMB))
    src = edge_index[0].astype(jnp.int32)
    dst = edge_index[1].astype(jnp.int32)
    partials = _scatter_kernel(h, src, dst)
    return _combine(partials)


# speedup vs baseline: 9.6471x; 1.7832x over previous
import functools

import jax
import jax.numpy as jnp
from jax import lax
from jax.experimental import pallas as pl
from jax.experimental.pallas import tpu as pltpu
from jax.experimental.pallas import tpu_sc as plsc

N_NODES = 10000
N_EDGES = 320000
EMB = 128

NC = 2
NS = 16
NW = NC * NS

EPT = N_EDGES // NW
K = 100
NCHUNK = EPT // K
WB = 80
N_PAD = 10240
RPT = N_PAD // NS


def _mlp_body(x_ref, w1_ref, b1_ref, w2_ref, b2_ref, o_ref):
    h = jnp.dot(x_ref[...], w1_ref[...], preferred_element_type=jnp.float32)
    h = jnp.maximum(h + b1_ref[...], 0.0)
    h = jnp.dot(h, w2_ref[...], preferred_element_type=jnp.float32)
    o_ref[...] = jnp.maximum(h + b2_ref[...], 0.0)


def _mlp(X, W1, b1, W2, b2):
    blk = 1000
    grid = (N_NODES // blk,)
    return pl.pallas_call(
        _mlp_body,
        grid=grid,
        in_specs=[
            pl.BlockSpec((blk, EMB), lambda i: (i, 0)),
            pl.BlockSpec((EMB, EMB), lambda i: (0, 0)),
            pl.BlockSpec((1, EMB), lambda i: (0, 0)),
            pl.BlockSpec((EMB, EMB), lambda i: (0, 0)),
            pl.BlockSpec((1, EMB), lambda i: (0, 0)),
        ],
        out_specs=pl.BlockSpec((blk, EMB), lambda i: (i, 0)),
        out_shape=jax.ShapeDtypeStruct((N_NODES, EMB), jnp.float32),
    )(X, W1, b1, W2, b2)


@functools.partial(
    pl.kernel,
    mesh=plsc.VectorSubcoreMesh(core_axis_name="c", subcore_axis_name="s"),
    out_type=jax.ShapeDtypeStruct((NC, N_PAD, EMB), jnp.float32),
    scratch_types=[
        pltpu.VMEM((2, K), jnp.int32),
        pltpu.VMEM((2, K), jnp.int32),
        pltpu.VMEM((K, EMB), jnp.float32),
        pltpu.VMEM((K, EMB), jnp.float32),
        pltpu.VMEM_SHARED((N_PAD, EMB), jnp.float32),
        pltpu.SemaphoreType.DMA,
        pltpu.SemaphoreType.DMA,
        pltpu.SemaphoreType.DMA,
        pltpu.SemaphoreType.DMA,
    ],
)
def _scatter_kernel(h_hbm, idx_hbm, out_hbm,
                    ib0, ib1, rows0, rows1, acc, si0, si1, sg0, sg1):
    c = lax.axis_index("c")
    s = lax.axis_index("s")
    wid = c * NS + s

    zero16 = jnp.zeros((16,), jnp.float32)

    def zero_row(r, carry):
        for j in range(EMB // 16):
            rows0[r, pl.ds(j * 16, 16)] = zero16
        return carry

    lax.fori_loop(0, WB, zero_row, 0)
    zsrc = rows0.at[pl.ds(0, WB)]
    for j in range(RPT // WB):
        pltpu.sync_copy(zsrc, acc.at[pl.ds(s * RPT + j * WB, WB)])
    plsc.subcore_barrier()

    last = NCHUNK - 1

    def start_idx(i, ib, sem):
        pltpu.async_copy(idx_hbm.at[wid, i], ib, sem)

    def wait_idx(ib, sem):
        pltpu.make_async_copy(idx_hbm.at[wid, 0], ib, sem).wait()

    def start_gather(ib, buf, sem):
        pltpu.async_copy(h_hbm.at[ib.at[0]], buf, sem)

    def wait_gather(ib, buf, sem):
        pltpu.make_async_copy(h_hbm.at[ib.at[0]], buf, sem).wait()

    def scatter(ib, buf):
        pltpu.sync_copy(buf, acc.at[ib.at[1]], add=True)

    start_idx(0, ib0, si0)
    start_idx(1, ib1, si1)
    wait_idx(ib0, si0)
    start_gather(ib0, rows0, sg0)
    wait_idx(ib1, si1)
    start_gather(ib1, rows1, sg1)

    def pair(g, carry):
        i0 = 2 * g
        i1 = i0 + 1
        wait_gather(ib0, rows0, sg0)
        scatter(ib0, rows0)
        start_idx(jnp.minimum(i0 + 2, last), ib0, si0)
        wait_gather(ib1, rows1, sg1)
        scatter(ib1, rows1)
        start_idx(jnp.minimum(i1 + 2, last), ib1, si1)
        wait_idx(ib0, si0)
        start_gather(ib0, rows0, sg0)
        wait_idx(ib1, si1)
        start_gather(ib1, rows1, sg1)
        return carry

    lax.fori_loop(0, NCHUNK // 2, pair, 0)
    wait_gather(ib0, rows0, sg0)
    wait_gather(ib1, rows1, sg1)
    plsc.subcore_barrier()

    for j in range(RPT // WB):
        r0 = s * RPT + j * WB
        pltpu.sync_copy(acc.at[pl.ds(r0, WB)], zsrc)
        pltpu.sync_copy(zsrc, out_hbm.at[c, pl.ds(r0, WB)])


def _add_body(p_ref, o_ref):
    o_ref[...] = p_ref[0] + p_ref[1]


def _combine(partials):
    blk = 1000
    grid = (N_NODES // blk,)
    return pl.pallas_call(
        _add_body,
        grid=grid,
        in_specs=[pl.BlockSpec((NC, blk, EMB), lambda i: (0, i, 0))],
        out_specs=pl.BlockSpec((blk, EMB), lambda i: (i, 0)),
        out_shape=jax.ShapeDtypeStruct((N_NODES, EMB), jnp.float32),
    )(partials)


def kernel(X, edge_index, W1, b1, W2, b2):
    h = _mlp(X, W1, b1.reshape(1, EMB), W2, b2.reshape(1, EMB))
    idx = jnp.stack(
        [edge_index[0].astype(jnp.int32).reshape(NW, NCHUNK, K),
         edge_index[1].astype(jnp.int32).reshape(NW, NCHUNK, K)], axis=2)
    partials = _scatter_kernel(h, idx)
    return _combine(partials)


# speedup vs baseline: 11.0515x; 1.1456x over previous
import functools

import jax
import jax.numpy as jnp
from jax import lax
from jax.experimental import pallas as pl
from jax.experimental.pallas import tpu as pltpu
from jax.experimental.pallas import tpu_sc as plsc

N_NODES = 10000
N_EDGES = 320000
EMB = 128

NC = 2
NS = 16
NW = NC * NS

EPT = N_EDGES // NW
K = 100
NCHUNK = EPT // K
WB = 80
N_PAD = 10240
RPT = N_PAD // NS


def _mlp_body(x_ref, w1_ref, b1_ref, w2_ref, b2_ref, o_ref):
    h = jnp.dot(x_ref[...], w1_ref[...], preferred_element_type=jnp.float32)
    h = jnp.maximum(h + b1_ref[...], 0.0)
    h = jnp.dot(h, w2_ref[...], preferred_element_type=jnp.float32)
    o_ref[...] = jnp.maximum(h + b2_ref[...], 0.0)


def _mlp(X, W1, b1, W2, b2):
    blk = 1000
    grid = (N_NODES // blk,)
    return pl.pallas_call(
        _mlp_body,
        grid=grid,
        in_specs=[
            pl.BlockSpec((blk, EMB), lambda i: (i, 0)),
            pl.BlockSpec((EMB, EMB), lambda i: (0, 0)),
            pl.BlockSpec((1, EMB), lambda i: (0, 0)),
            pl.BlockSpec((EMB, EMB), lambda i: (0, 0)),
            pl.BlockSpec((1, EMB), lambda i: (0, 0)),
        ],
        out_specs=pl.BlockSpec((blk, EMB), lambda i: (i, 0)),
        out_shape=jax.ShapeDtypeStruct((N_NODES, EMB), jnp.float32),
    )(X, W1, b1, W2, b2)


@functools.partial(
    pl.kernel,
    mesh=plsc.VectorSubcoreMesh(core_axis_name="c", subcore_axis_name="s"),
    out_type=jax.ShapeDtypeStruct((NC, N_PAD, EMB), jnp.float32),
    scratch_types=[
        pltpu.VMEM((2, 2, K), jnp.int32),
        pltpu.VMEM((2, 2, K), jnp.int32),
        pltpu.VMEM((K, EMB), jnp.float32),
        pltpu.VMEM((K, EMB), jnp.float32),
        pltpu.VMEM_SHARED((N_PAD, EMB), jnp.float32),
        pltpu.SemaphoreType.DMA,
        pltpu.SemaphoreType.DMA,
        pltpu.SemaphoreType.DMA,
        pltpu.SemaphoreType.DMA,
    ],
)
def _scatter_kernel(h_hbm, idx_hbm, out_hbm,
                    pxA, pxB, rows0, rows1, acc, siA, siB, sg0, sg1):
    c = lax.axis_index("c")
    s = lax.axis_index("s")
    wid = c * NS + s

    zero16 = jnp.zeros((16,), jnp.float32)

    def zero_row(r, carry):
        for j in range(EMB // 16):
            rows0[r, pl.ds(j * 16, 16)] = zero16
        return carry

    lax.fori_loop(0, WB, zero_row, 0)
    zsrc = rows0.at[pl.ds(0, WB)]
    for j in range(RPT // WB):
        pltpu.sync_copy(zsrc, acc.at[pl.ds(s * RPT + j * WB, WB)])
    plsc.subcore_barrier()

    last_pair = NCHUNK - 2

    def start_pair(i, px, sem):
        pltpu.async_copy(idx_hbm.at[wid, pl.ds(i, 2)], px, sem)

    def wait_pair(px, sem):
        pltpu.make_async_copy(idx_hbm.at[wid, pl.ds(0, 2)], px, sem).wait()

    def start_gather(isrc, buf, sem):
        pltpu.async_copy(h_hbm.at[isrc], buf, sem)

    def wait_gather(isrc, buf, sem):
        pltpu.make_async_copy(h_hbm.at[isrc], buf, sem).wait()

    def scatter(idst, buf):
        pltpu.sync_copy(buf, acc.at[idst], add=True)

    start_pair(0, pxA, siA)
    start_pair(2, pxB, siB)
    wait_pair(pxA, siA)
    start_gather(pxA.at[0, 0], rows0, sg0)
    start_gather(pxA.at[1, 0], rows1, sg1)

    def quad(g, carry):
        q = 4 * g
        wait_pair(pxB, siB)
        wait_gather(pxA.at[0, 0], rows0, sg0)
        scatter(pxA.at[0, 1], rows0)
        start_gather(pxB.at[0, 0], rows0, sg0)
        wait_gather(pxA.at[1, 0], rows1, sg1)
        scatter(pxA.at[1, 1], rows1)
        start_gather(pxB.at[1, 0], rows1, sg1)
        start_pair(jnp.minimum(q + 4, last_pair), pxA, siA)
        wait_gather(pxB.at[0, 0], rows0, sg0)
        scatter(pxB.at[0, 1], rows0)
        wait_pair(pxA, siA)
        start_gather(pxA.at[0, 0], rows0, sg0)
        wait_gather(pxB.at[1, 0], rows1, sg1)
        scatter(pxB.at[1, 1], rows1)
        start_gather(pxA.at[1, 0], rows1, sg1)
        start_pair(jnp.minimum(q + 6, last_pair), pxB, siB)
        return carry

    lax.fori_loop(0, NCHUNK // 4, quad, 0)
    wait_gather(pxA.at[0, 0], rows0, sg0)
    wait_gather(pxA.at[1, 0], rows1, sg1)
    wait_pair(pxB, siB)
    plsc.subcore_barrier()

    for j in range(RPT // WB):
        r0 = s * RPT + j * WB
        pltpu.sync_copy(acc.at[pl.ds(r0, WB)], zsrc)
        pltpu.sync_copy(zsrc, out_hbm.at[c, pl.ds(r0, WB)])


def _add_body(p_ref, o_ref):
    o_ref[...] = p_ref[0] + p_ref[1]


def _combine(partials):
    blk = 1000
    grid = (N_NODES // blk,)
    return pl.pallas_call(
        _add_body,
        grid=grid,
        in_specs=[pl.BlockSpec((NC, blk, EMB), lambda i: (0, i, 0))],
        out_specs=pl.BlockSpec((blk, EMB), lambda i: (i, 0)),
        out_shape=jax.ShapeDtypeStruct((N_NODES, EMB), jnp.float32),
    )(partials)


def kernel(X, edge_index, W1, b1, W2, b2):
    h = _mlp(X, W1, b1.reshape(1, EMB), W2, b2.reshape(1, EMB))
    idx = jnp.stack(
        [edge_index[0].astype(jnp.int32).reshape(NW, NCHUNK, K),
         edge_index[1].astype(jnp.int32).reshape(NW, NCHUNK, K)], axis=2)
    partials = _scatter_kernel(h, idx)
    return _combine(partials)
